# baseline (speedup 1.0000x reference)
import functools

import jax
import jax.numpy as jnp
from jax import lax
from jax.experimental import pallas as pl
from jax.experimental.pallas import tpu as pltpu
from jax.experimental.pallas import tpu_sc as plsc

N = 10000
E = 320000
NC = 2
NS = 16
NW = NC * NS
CHUNK = 100
EPT = E // NW
NCHK = EPT // CHUNK
NBUF = 5
RPS = 632
NPAD = RPS * NS
DEGW = 8

_MESH = plsc.VectorSubcoreMesh(core_axis_name="c", subcore_axis_name="s")
_SC_PARAMS = pltpu.CompilerParams(use_tc_tiling_on_sc=False)


@functools.partial(
    pl.kernel,
    out_type=jax.ShapeDtypeStruct((NC, NPAD, DEGW), jnp.float32),
    mesh=_MESH,
    compiler_params=_SC_PARAMS,
    scratch_types=[
        pltpu.VMEM((NCHK, CHUNK), jnp.int32),
        pltpu.VMEM((CHUNK, DEGW), jnp.float32),
        pltpu.VMEM_SHARED((NPAD, DEGW), jnp.float32),
        pltpu.SemaphoreType.DMA,
    ],
)
def _deg(ei_hbm, ones_hbm, zeros_hbm, out_hbm, dst_v, ones_v, acc_sh, sem):
    c = lax.axis_index("c")
    s = lax.axis_index("s")
    w = c * NS + s
    pltpu.sync_copy(ei_hbm.at[1, w], dst_v)
    pltpu.sync_copy(ones_hbm, ones_v)
    pltpu.sync_copy(zeros_hbm, acc_sh.at[pl.ds(s * RPS, RPS)])
    plsc.subcore_barrier()

    def fire(j, carry):
        pltpu.async_copy(ones_v, acc_sh.at[dst_v.at[j]], sem, add=True)
        return carry

    lax.fori_loop(0, NCHK, fire, 0)

    def drain(j, carry):
        pltpu.make_async_copy(ones_v, acc_sh.at[dst_v.at[0]], sem).wait()
        return carry

    lax.fori_loop(0, NCHK, drain, 0)
    plsc.subcore_barrier()
    pltpu.sync_copy(acc_sh.at[pl.ds(s * RPS, RPS)],
                    out_hbm.at[c, pl.ds(s * RPS, RPS)])


def _make_conv(D):
    @functools.partial(
        pl.kernel,
        out_type=jax.ShapeDtypeStruct((NC, NPAD, D), jnp.float32),
        mesh=_MESH,
        compiler_params=_SC_PARAMS,
        scratch_types=[
            pltpu.VMEM((NCHK, CHUNK), jnp.int32),
            pltpu.VMEM((NCHK, CHUNK), jnp.int32),
            [pltpu.VMEM((CHUNK, D), jnp.float32)] * NBUF,
            pltpu.VMEM_SHARED((NPAD, D), jnp.float32),
            [pltpu.SemaphoreType.DMA] * NBUF,
            [pltpu.SemaphoreType.DMA] * NBUF,
        ],
    )
    def conv(table_hbm, ei_hbm, zeros_hbm, out_hbm,
             src_v, dst_v, rows, acc_sh, gsem, ssem):
        c = lax.axis_index("c")
        s = lax.axis_index("s")
        w = c * NS + s
        pltpu.sync_copy(ei_hbm.at[0, w], src_v)
        pltpu.sync_copy(ei_hbm.at[1, w], dst_v)
        pltpu.sync_copy(zeros_hbm, acc_sh.at[pl.ds(s * RPS, RPS)])
        plsc.subcore_barrier()

        for b in range(NBUF):
            pltpu.async_copy(table_hbm.at[src_v.at[b]], rows[b], gsem[b])

        niter = NCHK // NBUF

        def body(jj, carry):
            j = jj * NBUF
            for b in range(NBUF):
                pltpu.make_async_copy(
                    table_hbm.at[src_v.at[j + b]], rows[b], gsem[b]).wait()
                pltpu.async_copy(
                    rows[b], acc_sh.at[dst_v.at[j + b]], ssem[b], add=True)

            @pl.when(jj + 1 < niter)
            def _():
                for b in range(NBUF):
                    pltpu.make_async_copy(
                        rows[b], acc_sh.at[dst_v.at[j + b]], ssem[b]).wait()
                    pltpu.async_copy(
                        table_hbm.at[src_v.at[j + b + NBUF]], rows[b],
                        gsem[b])

            return carry

        lax.fori_loop(0, niter, body, 0)
        for b in range(NBUF):
            pltpu.make_async_copy(
                rows[b], acc_sh.at[dst_v.at[NCHK - NBUF + b]], ssem[b]).wait()
        plsc.subcore_barrier()
        pltpu.sync_copy(acc_sh.at[pl.ds(s * RPS, RPS)],
                        out_hbm.at[c, pl.ds(s * RPS, RPS)])

    return conv


_conv64 = _make_conv(64)
_conv32 = _make_conv(32)


@functools.partial(
    pl.kernel,
    out_type=jax.ShapeDtypeStruct((E, 32), jnp.float32),
    mesh=_MESH,
    compiler_params=_SC_PARAMS,
    scratch_types=[
        pltpu.VMEM((NCHK, CHUNK), jnp.int32),
        pltpu.VMEM((NCHK, CHUNK), jnp.int32),
        [pltpu.VMEM((CHUNK, 32), jnp.float32)] * 2,
        [pltpu.VMEM((CHUNK, 32), jnp.float32)] * 2,
        [pltpu.VMEM((CHUNK, 32), jnp.float32)] * 2,
        [pltpu.SemaphoreType.DMA] * 2,
        [pltpu.SemaphoreType.DMA] * 2,
        [pltpu.SemaphoreType.DMA] * 2,
    ],
)
def _dec(tab_hbm, eli_hbm, out_hbm,
         ia_v, ib_v, abuf, bbuf, pbuf, gasem, gbsem, wsem):
    c = lax.axis_index("c")
    s = lax.axis_index("s")
    w = c * NS + s
    base = w * EPT
    pltpu.sync_copy(eli_hbm.at[0, w], ia_v)
    pltpu.sync_copy(eli_hbm.at[1, w], ib_v)

    for b in range(2):
        pltpu.async_copy(tab_hbm.at[ia_v.at[b]], abuf[b], gasem[b])
        pltpu.async_copy(tab_hbm.at[ib_v.at[b]], bbuf[b], gbsem[b])

    niter = NCHK // 2

    def body(jj, carry):
        j = jj * 2
        for b in range(2):
            pltpu.make_async_copy(
                tab_hbm.at[ia_v.at[j + b]], abuf[b], gasem[b]).wait()
            pltpu.make_async_copy(
                tab_hbm.at[ib_v.at[j + b]], bbuf[b], gbsem[b]).wait()

            @pl.when(jj > 0)
            def _():
                pltpu.make_async_copy(
                    pbuf[b], out_hbm.at[pl.ds(base, CHUNK)], wsem[b]).wait()

            @plsc.parallel_loop(0, CHUNK, unroll=8)
            def inner(r):
                pbuf[b][r, pl.ds(0, 16)] = (abuf[b][r, pl.ds(0, 16)]
                                            * bbuf[b][r, pl.ds(0, 16)])
                pbuf[b][r, pl.ds(16, 16)] = (abuf[b][r, pl.ds(16, 16)]
                                             * bbuf[b][r, pl.ds(16, 16)])
            pltpu.async_copy(
                pbuf[b], out_hbm.at[pl.ds(base + (j + b) * CHUNK, CHUNK)],
                wsem[b])

            @pl.when(jj + 1 < niter)
            def _():
                pltpu.async_copy(tab_hbm.at[ia_v.at[j + b + 2]], abuf[b],
                                 gasem[b])
                pltpu.async_copy(tab_hbm.at[ib_v.at[j + b + 2]], bbuf[b],
                                 gbsem[b])

        return carry

    lax.fori_loop(0, niter, body, 0)
    for b in range(2):
        pltpu.make_async_copy(
            pbuf[b], out_hbm.at[pl.ds(base, CHUNK)], wsem[b]).wait()


def _pre_body(x_ref, w1, b1, w2, b2, wc1, p0_3, p1_3, hws_o, dinv_o):
    p0, p1 = p0_3.at[0], p1_3.at[0]
    h = jnp.dot(x_ref[...], w1[...], preferred_element_type=jnp.float32)
    h = jnp.maximum(h + b1[...], 0.0)
    h = jnp.dot(h, w2[...], preferred_element_type=jnp.float32)
    h = jnp.maximum(h + b2[...], 0.0)
    hw1 = jnp.dot(h, wc1[...], preferred_element_type=jnp.float32)
    deg8 = 1.0 + p0[...] + p1[...]
    dinv8 = lax.rsqrt(deg8)
    dinv64 = jnp.broadcast_to(dinv8[:, 0:1], (hw1.shape[0], 64))
    dinv_o[...] = dinv64
    hws_o[...] = hw1 * dinv64


def _pre_mlp(x, W1, b1, W2, b2, Wc1, degp):
    blk = 1000
    return pl.pallas_call(
        _pre_body,
        grid=(N // blk,),
        in_specs=[
            pl.BlockSpec((blk, 128), lambda i: (i, 0)),
            pl.BlockSpec((128, 256), lambda i: (0, 0)),
            pl.BlockSpec((1, 256), lambda i: (0, 0)),
            pl.BlockSpec((256, 128), lambda i: (0, 0)),
            pl.BlockSpec((1, 128), lambda i: (0, 0)),
            pl.BlockSpec((128, 64), lambda i: (0, 0)),
            pl.BlockSpec((1, blk, DEGW), lambda i: (0, i, 0)),
            pl.BlockSpec((1, blk, DEGW), lambda i: (1, i, 0)),
        ],
        out_specs=(
            pl.BlockSpec((blk, 64), lambda i: (i, 0)),
            pl.BlockSpec((blk, 64), lambda i: (i, 0)),
        ),
        out_shape=(
            jax.ShapeDtypeStruct((N, 64), jnp.float32),
            jax.ShapeDtypeStruct((N, 64), jnp.float32),
        ),
    )(x, W1, b1, W2, b2, Wc1, degp, degp)


def _mid_body(q0_3, q1_3, hws1, dinv, b1, wc2, emb1_o, hws2_o):
    q0, q1 = q0_3.at[0], q1_3.at[0]
    dv = dinv[...]
    pre = dv * (q0[...] + q1[...] + hws1[...]) + b1[...]
    emb1 = jnp.maximum(pre, 0.0)
    emb1_o[...] = emb1
    hw2 = jnp.dot(emb1, wc2[...], preferred_element_type=jnp.float32)
    hws2_o[...] = hw2 * dv[:, :32]


def _mid(acc, hws1, dinv, b1, Wc2):
    return pl.pallas_call(
        _mid_body,
        grid=(1,),
        in_specs=[
            pl.BlockSpec((1, N, 64), lambda i: (0, 0, 0)),
            pl.BlockSpec((1, N, 64), lambda i: (1, 0, 0)),
            pl.BlockSpec((N, 64), lambda i: (0, 0)),
            pl.BlockSpec((N, 64), lambda i: (0, 0)),
            pl.BlockSpec((1, 64), lambda i: (0, 0)),
            pl.BlockSpec((64, 32), lambda i: (0, 0)),
        ],
        out_specs=(
            pl.BlockSpec((N, 64), lambda i: (0, 0)),
            pl.BlockSpec((N, 32), lambda i: (0, 0)),
        ),
        out_shape=(
            jax.ShapeDtypeStruct((N, 64), jnp.float32),
            jax.ShapeDtypeStruct((N, 32), jnp.float32),
        ),
    )(acc, acc, hws1, dinv, b1, Wc2)


def _post_body(q0_3, q1_3, hws2, dinv, b2, emb2_o):
    q0, q1 = q0_3.at[0], q1_3.at[0]
    dv = dinv[:, :32]
    pre = dv * (q0[...] + q1[...] + hws2[...]) + b2[...]
    emb2_o[...] = jnp.maximum(pre, 0.0)


def _post(acc, hws2, dinv, b2):
    return pl.pallas_call(
        _post_body,
        grid=(1,),
        in_specs=[
            pl.BlockSpec((1, N, 32), lambda i: (0, 0, 0)),
            pl.BlockSpec((1, N, 32), lambda i: (1, 0, 0)),
            pl.BlockSpec((N, 32), lambda i: (0, 0)),
            pl.BlockSpec((N, 64), lambda i: (0, 0)),
            pl.BlockSpec((1, 32), lambda i: (0, 0)),
        ],
        out_specs=pl.BlockSpec((N, 32), lambda i: (0, 0)),
        out_shape=jax.ShapeDtypeStruct((N, 32), jnp.float32),
    )(acc, acc, hws2, dinv, b2)


def _red_body(p_ref, wb_ref, bp_ref, o_ref):
    cst = jnp.sum(bp_ref[...])
    blk4 = p_ref.shape[0] // 128
    p4 = p_ref[...].reshape(blk4, 128)
    d = jnp.dot(p4, wb_ref[...])
    for q in range(4):
        o_ref[0, q, :] = d[:, 2 * q] + d[:, 2 * q + 1] + cst
    for q in range(4, 8):
        o_ref[0, q, :] = jnp.zeros((blk4,), jnp.float32)


def _red(p1d, wb, bp):
    blk4 = 3200
    nblk = E // (4 * blk4)
    return pl.pallas_call(
        _red_body,
        grid=(nblk,),
        in_specs=[
            pl.BlockSpec((blk4 * 128,), lambda i: (i,)),
            pl.BlockSpec((128, 8), lambda i: (0, 0)),
            pl.BlockSpec((1, 2), lambda i: (0, 0)),
        ],
        out_specs=pl.BlockSpec((1, 8, blk4), lambda i: (i, 0, 0)),
        out_shape=jax.ShapeDtypeStruct((nblk, 8, blk4), jnp.float32),
    )(p1d, wb, bp)


def kernel(x, edge_index, edge_label_index, W_pre1, b_pre1, W_pre2, b_pre2,
           W_conv1, b_conv1, W_conv2, b_conv2, W_post, b_post):
    ei_g = edge_index.reshape(2, NW, NCHK, CHUNK)
    eli_g = edge_label_index.reshape(2, NW, NCHK, CHUNK)

    ones8 = jnp.ones((CHUNK, DEGW), jnp.float32)
    zeros8 = jnp.zeros((RPS, DEGW), jnp.float32)
    zeros64 = jnp.zeros((RPS, 64), jnp.float32)
    zeros32 = jnp.zeros((RPS, 32), jnp.float32)

    degp = _deg(ei_g, ones8, zeros8)

    hws1, dinv = _pre_mlp(x, W_pre1, b_pre1.reshape(1, 256), W_pre2,
                          b_pre2.reshape(1, 128), W_conv1, degp)

    acc1 = _conv64(hws1, ei_g, zeros64)

    emb1, hws2 = _mid(acc1, hws1, dinv,
                      b_conv1.reshape(1, 64), W_conv2)

    acc2 = _conv32(hws2, ei_g, zeros32)

    emb2 = _post(acc2, hws2, dinv, b_conv2.reshape(1, 32))

    prod = _dec(emb2, eli_g)

    wb = jnp.zeros((128, 8), jnp.float32)
    for q in range(4):
        wb = wb.at[32 * q:32 * (q + 1), 2 * q:2 * (q + 1)].set(W_post)
    s_q = _red(prod.reshape(E * 32), wb, b_post.reshape(1, 2))
    score = s_q[:, :4, :].transpose(0, 2, 1).reshape(E)

    return (score, emb1, emb2)


# speedup vs baseline: 1.1621x; 1.1621x over previous
import functools

import jax
import jax.numpy as jnp
from jax import lax
from jax.experimental import pallas as pl
from jax.experimental.pallas import tpu as pltpu
from jax.experimental.pallas import tpu_sc as plsc

N = 10000
E = 320000
NC = 2
NS = 16
NW = NC * NS
CHUNK = 100
EPT = E // NW
NCHK = EPT // CHUNK
NBUF = 5
RPS = 632
NPAD = RPS * NS
DEGW = 8

_MESH = plsc.VectorSubcoreMesh(core_axis_name="c", subcore_axis_name="s")
_SC_PARAMS = pltpu.CompilerParams(use_tc_tiling_on_sc=False)


@functools.partial(
    pl.kernel,
    out_type=jax.ShapeDtypeStruct((NC, NPAD, DEGW), jnp.float32),
    mesh=_MESH,
    compiler_params=_SC_PARAMS,
    scratch_types=[
        pltpu.VMEM((NCHK, CHUNK), jnp.int32),
        pltpu.VMEM((CHUNK, DEGW), jnp.float32),
        pltpu.VMEM_SHARED((NPAD, DEGW), jnp.float32),
        pltpu.SemaphoreType.DMA,
    ],
)
def _deg(ei_hbm, ones_hbm, zeros_hbm, out_hbm, dst_v, ones_v, acc_sh, sem):
    c = lax.axis_index("c")
    s = lax.axis_index("s")
    w = c * NS + s
    pltpu.sync_copy(ei_hbm.at[1, w], dst_v)
    pltpu.sync_copy(ones_hbm, ones_v)
    pltpu.sync_copy(zeros_hbm, acc_sh.at[pl.ds(s * RPS, RPS)])
    plsc.subcore_barrier()

    def fire(j, carry):
        pltpu.async_copy(ones_v, acc_sh.at[dst_v.at[j]], sem, add=True)
        return carry

    lax.fori_loop(0, NCHK, fire, 0)

    def drain(j, carry):
        pltpu.make_async_copy(ones_v, acc_sh.at[dst_v.at[0]], sem).wait()
        return carry

    lax.fori_loop(0, NCHK, drain, 0)
    plsc.subcore_barrier()
    pltpu.sync_copy(acc_sh.at[pl.ds(s * RPS, RPS)],
                    out_hbm.at[c, pl.ds(s * RPS, RPS)])


def _make_conv(D):
    @functools.partial(
        pl.kernel,
        out_type=jax.ShapeDtypeStruct((NC, NPAD, D), jnp.float32),
        mesh=_MESH,
        compiler_params=_SC_PARAMS,
        scratch_types=[
            pltpu.VMEM((NCHK, CHUNK), jnp.int32),
            pltpu.VMEM((NCHK, CHUNK), jnp.int32),
            [pltpu.VMEM((CHUNK, D), jnp.float32)] * NBUF,
            pltpu.VMEM_SHARED((NPAD, D), jnp.float32),
            [pltpu.SemaphoreType.DMA] * NBUF,
            [pltpu.SemaphoreType.DMA] * NBUF,
        ],
    )
    def conv(table_hbm, ei_hbm, zeros_hbm, out_hbm,
             src_v, dst_v, rows, acc_sh, gsem, ssem):
        c = lax.axis_index("c")
        s = lax.axis_index("s")
        w = c * NS + s
        pltpu.sync_copy(ei_hbm.at[0, w], src_v)
        pltpu.sync_copy(ei_hbm.at[1, w], dst_v)
        pltpu.sync_copy(zeros_hbm, acc_sh.at[pl.ds(s * RPS, RPS)])
        plsc.subcore_barrier()

        for b in range(NBUF):
            pltpu.async_copy(table_hbm.at[src_v.at[b]], rows[b], gsem[b])

        niter = NCHK // NBUF

        def body(jj, carry):
            j = jj * NBUF
            for b in range(NBUF):
                pltpu.make_async_copy(
                    table_hbm.at[src_v.at[j + b]], rows[b], gsem[b]).wait()
                pltpu.async_copy(
                    rows[b], acc_sh.at[dst_v.at[j + b]], ssem[b], add=True)

            @pl.when(jj + 1 < niter)
            def _():
                for b in range(NBUF):
                    pltpu.make_async_copy(
                        rows[b], acc_sh.at[dst_v.at[j + b]], ssem[b]).wait()
                    pltpu.async_copy(
                        table_hbm.at[src_v.at[j + b + NBUF]], rows[b],
                        gsem[b])

            return carry

        lax.fori_loop(0, niter, body, 0)
        for b in range(NBUF):
            pltpu.make_async_copy(
                rows[b], acc_sh.at[dst_v.at[NCHK - NBUF + b]], ssem[b]).wait()
        plsc.subcore_barrier()
        pltpu.sync_copy(acc_sh.at[pl.ds(s * RPS, RPS)],
                        out_hbm.at[c, pl.ds(s * RPS, RPS)])

    return conv


_conv64 = _make_conv(64)
_conv32 = _make_conv(32)


@functools.partial(
    pl.kernel,
    out_type=jax.ShapeDtypeStruct((E, 32), jnp.float32),
    mesh=_MESH,
    compiler_params=_SC_PARAMS,
    scratch_types=[
        pltpu.VMEM((NCHK, CHUNK), jnp.int32),
        pltpu.VMEM((NCHK, CHUNK), jnp.int32),
        [pltpu.VMEM((CHUNK, 32), jnp.float32)] * 4,
        [pltpu.VMEM((CHUNK, 32), jnp.float32)] * 4,
        [pltpu.VMEM((CHUNK, 32), jnp.float32)] * 4,
        [pltpu.SemaphoreType.DMA] * 4,
        [pltpu.SemaphoreType.DMA] * 4,
        [pltpu.SemaphoreType.DMA] * 4,
    ],
)
def _dec(tab_hbm, eli_hbm, out_hbm,
         ia_v, ib_v, abuf, bbuf, pbuf, gasem, gbsem, wsem):
    c = lax.axis_index("c")
    s = lax.axis_index("s")
    w = c * NS + s
    base = w * EPT
    pltpu.sync_copy(eli_hbm.at[0, w], ia_v)
    pltpu.sync_copy(eli_hbm.at[1, w], ib_v)

    for b in range(4):
        pltpu.async_copy(tab_hbm.at[ia_v.at[b]], abuf[b], gasem[b])
        pltpu.async_copy(tab_hbm.at[ib_v.at[b]], bbuf[b], gbsem[b])

    niter = NCHK // 4

    def body(jj, carry):
        j = jj * 4
        for b in range(4):
            pltpu.make_async_copy(
                tab_hbm.at[ia_v.at[j + b]], abuf[b], gasem[b]).wait()
            pltpu.make_async_copy(
                tab_hbm.at[ib_v.at[j + b]], bbuf[b], gbsem[b]).wait()

            @pl.when(jj > 0)
            def _():
                pltpu.make_async_copy(
                    pbuf[b], out_hbm.at[pl.ds(base, CHUNK)], wsem[b]).wait()

            @plsc.parallel_loop(0, CHUNK, unroll=8)
            def inner(r):
                pbuf[b][r, pl.ds(0, 16)] = (abuf[b][r, pl.ds(0, 16)]
                                            * bbuf[b][r, pl.ds(0, 16)])
                pbuf[b][r, pl.ds(16, 16)] = (abuf[b][r, pl.ds(16, 16)]
                                             * bbuf[b][r, pl.ds(16, 16)])
            pltpu.async_copy(
                pbuf[b], out_hbm.at[pl.ds(base + (j + b) * CHUNK, CHUNK)],
                wsem[b])

            @pl.when(jj + 1 < niter)
            def _():
                pltpu.async_copy(tab_hbm.at[ia_v.at[j + b + 4]], abuf[b],
                                 gasem[b])
                pltpu.async_copy(tab_hbm.at[ib_v.at[j + b + 4]], bbuf[b],
                                 gbsem[b])

        return carry

    lax.fori_loop(0, niter, body, 0)
    for b in range(4):
        pltpu.make_async_copy(
            pbuf[b], out_hbm.at[pl.ds(base, CHUNK)], wsem[b]).wait()


def _pre_body(x_ref, w1, b1, w2, b2, wc1, p0_3, p1_3, hws_o, dinv_o):
    p0, p1 = p0_3.at[0], p1_3.at[0]
    h = jnp.dot(x_ref[...], w1[...], preferred_element_type=jnp.float32)
    h = jnp.maximum(h + b1[...], 0.0)
    h = jnp.dot(h, w2[...], preferred_element_type=jnp.float32)
    h = jnp.maximum(h + b2[...], 0.0)
    hw1 = jnp.dot(h, wc1[...], preferred_element_type=jnp.float32)
    deg8 = 1.0 + p0[...] + p1[...]
    dinv8 = lax.rsqrt(deg8)
    dinv64 = jnp.broadcast_to(dinv8[:, 0:1], (hw1.shape[0], 64))
    dinv_o[...] = dinv64
    hws_o[...] = hw1 * dinv64


def _pre_mlp(x, W1, b1, W2, b2, Wc1, degp):
    blk = 1000
    return pl.pallas_call(
        _pre_body,
        grid=(N // blk,),
        in_specs=[
            pl.BlockSpec((blk, 128), lambda i: (i, 0)),
            pl.BlockSpec((128, 256), lambda i: (0, 0)),
            pl.BlockSpec((1, 256), lambda i: (0, 0)),
            pl.BlockSpec((256, 128), lambda i: (0, 0)),
            pl.BlockSpec((1, 128), lambda i: (0, 0)),
            pl.BlockSpec((128, 64), lambda i: (0, 0)),
            pl.BlockSpec((1, blk, DEGW), lambda i: (0, i, 0)),
            pl.BlockSpec((1, blk, DEGW), lambda i: (1, i, 0)),
        ],
        out_specs=(
            pl.BlockSpec((blk, 64), lambda i: (i, 0)),
            pl.BlockSpec((blk, 64), lambda i: (i, 0)),
        ),
        out_shape=(
            jax.ShapeDtypeStruct((N, 64), jnp.float32),
            jax.ShapeDtypeStruct((N, 64), jnp.float32),
        ),
    )(x, W1, b1, W2, b2, Wc1, degp, degp)


def _mid_body(q0_3, q1_3, hws1, dinv, b1, wc2, emb1_o, hws2_o):
    q0, q1 = q0_3.at[0], q1_3.at[0]
    dv = dinv[...]
    pre = dv * (q0[...] + q1[...] + hws1[...]) + b1[...]
    emb1 = jnp.maximum(pre, 0.0)
    emb1_o[...] = emb1
    hw2 = jnp.dot(emb1, wc2[...], preferred_element_type=jnp.float32)
    hws2_o[...] = hw2 * dv[:, :32]


def _mid(acc, hws1, dinv, b1, Wc2):
    return pl.pallas_call(
        _mid_body,
        grid=(1,),
        in_specs=[
            pl.BlockSpec((1, N, 64), lambda i: (0, 0, 0)),
            pl.BlockSpec((1, N, 64), lambda i: (1, 0, 0)),
            pl.BlockSpec((N, 64), lambda i: (0, 0)),
            pl.BlockSpec((N, 64), lambda i: (0, 0)),
            pl.BlockSpec((1, 64), lambda i: (0, 0)),
            pl.BlockSpec((64, 32), lambda i: (0, 0)),
        ],
        out_specs=(
            pl.BlockSpec((N, 64), lambda i: (0, 0)),
            pl.BlockSpec((N, 32), lambda i: (0, 0)),
        ),
        out_shape=(
            jax.ShapeDtypeStruct((N, 64), jnp.float32),
            jax.ShapeDtypeStruct((N, 32), jnp.float32),
        ),
    )(acc, acc, hws1, dinv, b1, Wc2)


def _post_body(q0_3, q1_3, hws2, dinv, b2, emb2_o):
    q0, q1 = q0_3.at[0], q1_3.at[0]
    dv = dinv[:, :32]
    pre = dv * (q0[...] + q1[...] + hws2[...]) + b2[...]
    emb2_o[...] = jnp.maximum(pre, 0.0)


def _post(acc, hws2, dinv, b2):
    return pl.pallas_call(
        _post_body,
        grid=(1,),
        in_specs=[
            pl.BlockSpec((1, N, 32), lambda i: (0, 0, 0)),
            pl.BlockSpec((1, N, 32), lambda i: (1, 0, 0)),
            pl.BlockSpec((N, 32), lambda i: (0, 0)),
            pl.BlockSpec((N, 64), lambda i: (0, 0)),
            pl.BlockSpec((1, 32), lambda i: (0, 0)),
        ],
        out_specs=pl.BlockSpec((N, 32), lambda i: (0, 0)),
        out_shape=jax.ShapeDtypeStruct((N, 32), jnp.float32),
    )(acc, acc, hws2, dinv, b2)


def _red_body(p_ref, wb_ref, bp_ref, o_ref):
    cst = jnp.sum(bp_ref[...])
    blk4 = p_ref.shape[0] // 128
    p4 = p_ref[...].reshape(blk4, 128)
    d = jnp.dot(p4, wb_ref[...])
    s4 = jnp.concatenate(
        [d[:, 2 * q:2 * q + 1] + d[:, 2 * q + 1:2 * q + 2] for q in range(4)],
        axis=1) + cst
    o_ref[...] = s4[None]


def _red(p1d, wb, bp):
    blk4 = 3200
    nblk = E // (4 * blk4)
    return pl.pallas_call(
        _red_body,
        grid=(nblk,),
        in_specs=[
            pl.BlockSpec((blk4 * 128,), lambda i: (i,)),
            pl.BlockSpec((128, 8), lambda i: (0, 0)),
            pl.BlockSpec((1, 2), lambda i: (0, 0)),
        ],
        out_specs=pl.BlockSpec((1, blk4, 4), lambda i: (i, 0, 0)),
        out_shape=jax.ShapeDtypeStruct((nblk, blk4, 4), jnp.float32),
    )(p1d, wb, bp)


def kernel(x, edge_index, edge_label_index, W_pre1, b_pre1, W_pre2, b_pre2,
           W_conv1, b_conv1, W_conv2, b_conv2, W_post, b_post):
    ei_g = edge_index.reshape(2, NW, NCHK, CHUNK)
    eli_g = edge_label_index.reshape(2, NW, NCHK, CHUNK)

    ones8 = jnp.ones((CHUNK, DEGW), jnp.float32)
    zeros8 = jnp.zeros((RPS, DEGW), jnp.float32)
    zeros64 = jnp.zeros((RPS, 64), jnp.float32)
    zeros32 = jnp.zeros((RPS, 32), jnp.float32)

    degp = _deg(ei_g, ones8, zeros8)

    hws1, dinv = _pre_mlp(x, W_pre1, b_pre1.reshape(1, 256), W_pre2,
                          b_pre2.reshape(1, 128), W_conv1, degp)

    acc1 = _conv64(hws1, ei_g, zeros64)

    emb1, hws2 = _mid(acc1, hws1, dinv,
                      b_conv1.reshape(1, 64), W_conv2)

    acc2 = _conv32(hws2, ei_g, zeros32)

    emb2 = _post(acc2, hws2, dinv, b_conv2.reshape(1, 32))

    prod = _dec(emb2, eli_g)

    wb = jnp.zeros((128, 8), jnp.float32)
    for q in range(4):
        wb = wb.at[32 * q:32 * (q + 1), 2 * q:2 * (q + 1)].set(W_post)
    score = _red(prod.reshape(E * 32), wb,
                 b_post.reshape(1, 2)).reshape(E)

    return (score, emb1, emb2)
